# baseline (speedup 1.0000x reference)
import jax
import jax.numpy as jnp
from jax import lax
from jax.experimental import pallas as pl
from jax.experimental.pallas import tpu as pltpu
from jax.experimental.pallas import tpu_sc as plsc

VOCAB = 24
EMBED_DIM = 128
BATCH = 256
SEQ = 1024

NC = 2
NS = 16
NW = NC * NS

TOTAL = BATCH * SEQ
PER_W = TOTAL // NW
GROUP = 128
NGROUPS = PER_W // GROUP
NBUF = 4
LANES = 16
NCH = EMBED_DIM // LANES


def _emb_kernel(table_hbm, idx_hbm, out_hbm, table_v, idx_v, obuf, osem):
    wid = lax.axis_index("s") * NC + lax.axis_index("c")
    pltpu.sync_copy(table_hbm, table_v)
    pltpu.sync_copy(idx_hbm.at[wid], idx_v)
    base = wid * PER_W

    def wait_out():
        pltpu.make_async_copy(
            obuf.at[0], out_hbm.at[pl.ds(base, GROUP)], osem
        ).wait()

    def body(g, _):
        b = lax.rem(g, NBUF)

        @pl.when(g >= NBUF)
        def _():
            wait_out()

        def block(i, _):
            j0 = i * LANES
            idxv = idx_v[g, pl.ds(j0, LANES)]
            for l in range(LANES):
                idx = idxv[l]
                for k in range(NCH):
                    sl = pl.ds(k * LANES, LANES)
                    obuf[b, j0 + l, sl] = table_v[idx, sl]
            return 0

        lax.fori_loop(0, GROUP // LANES, block, 0)
        pltpu.async_copy(
            obuf.at[b], out_hbm.at[pl.ds(base + g * GROUP, GROUP)], osem
        )
        return 0

    lax.fori_loop(0, NGROUPS, body, 0)

    for _ in range(min(NBUF, NGROUPS)):
        wait_out()


def kernel(x, table):
    idx = x.reshape(NW, NGROUPS, GROUP)
    mesh = plsc.VectorSubcoreMesh(core_axis_name="c", subcore_axis_name="s")
    out = pl.kernel(
        _emb_kernel,
        mesh=mesh,
        out_type=jax.ShapeDtypeStruct((TOTAL, EMBED_DIM), jnp.float32),
        scratch_types=[
            pltpu.VMEM((VOCAB, EMBED_DIM), jnp.float32),
            pltpu.VMEM((NGROUPS, GROUP), jnp.int32),
            pltpu.VMEM((NBUF, GROUP, EMBED_DIM), jnp.float32),
            pltpu.SemaphoreType.DMA,
        ],
    )(table, idx)
    return out.reshape(BATCH, SEQ, EMBED_DIM)
